# Initial kernel scaffold; baseline (speedup 1.0000x reference)
import jax
import jax.numpy as jnp
from jax.experimental import pallas as pl


def kernel(logic_x, phys_x, Wl1, bl1, Wl2, bl2, Wp1, bp1, Wp2, bp2, Was, bas, Wap, bap, logic_edge_index, phys_edge_index):
    raise NotImplementedError("write your pallas kernel here")


# speedup vs baseline: 3.1080x; 3.1080x over previous
import functools

import jax
import jax.numpy as jnp
from jax import lax
from jax.experimental import pallas as pl
from jax.experimental.pallas import tpu as pltpu
from jax.experimental.pallas import tpu_sc as plsc

N = 10000
DIN = 128
EMB = 64
REG = 10240
TOT = 2 * REG
NC, NS = 2, 16
ROWS_PER_TILE = TOT // NS
LANE = 128
E_TOT = 320000 + 160000
CHUNKS_PER_TILE = -(-E_TOT // (NC * NS * LANE))
TOT_CHUNKS = NC * NS * CHUNKS_PER_TILE
E_PAD = TOT_CHUNKS * LANE
DEG_W = 16
SCALE = EMB ** (-0.5)


def _seg_body(y, srcc, dstc, zer64, agg_out, src_v, dst_v, rows_v, acc, sem,
              deg=None):
    cid = lax.axis_index("c")
    sid = lax.axis_index("s")
    base = (cid * NS + sid) * CHUNKS_PER_TILE
    row0 = sid * ROWS_PER_TILE
    pltpu.sync_copy(zer64, acc.at[pl.ds(row0, ROWS_PER_TILE)])
    if deg is not None:
        zer16, ones16, deg_out, ones_v, accd = deg
        pltpu.sync_copy(zer16, accd.at[pl.ds(row0, ROWS_PER_TILE)])
        pltpu.sync_copy(ones16, ones_v)
    plsc.subcore_barrier()

    def chunk(j, carry):
        pltpu.sync_copy(srcc.at[base + j], src_v)
        pltpu.sync_copy(dstc.at[base + j], dst_v)
        pltpu.async_copy(y.at[src_v], rows_v, sem).wait()
        pltpu.sync_copy(rows_v, acc.at[dst_v], add=True)
        if deg is not None:
            pltpu.sync_copy(ones_v, accd.at[dst_v], add=True)
        return carry

    lax.fori_loop(0, CHUNKS_PER_TILE, chunk, 0)
    plsc.subcore_barrier()
    pltpu.sync_copy(acc.at[pl.ds(row0, ROWS_PER_TILE)],
                    agg_out.at[cid, pl.ds(row0, ROWS_PER_TILE)])
    if deg is not None:
        pltpu.sync_copy(accd.at[pl.ds(row0, ROWS_PER_TILE)],
                        deg_out.at[cid, pl.ds(row0, ROWS_PER_TILE)])


def _make_seg_sum(with_deg):
    mesh = plsc.VectorSubcoreMesh(core_axis_name="c", subcore_axis_name="s")
    outs = [jax.ShapeDtypeStruct((NC, TOT, EMB), jnp.float32)]
    scratch = [
        pltpu.VMEM((LANE,), jnp.int32),
        pltpu.VMEM((LANE,), jnp.int32),
        pltpu.VMEM((LANE, EMB), jnp.float32),
        pltpu.VMEM_SHARED((TOT, EMB), jnp.float32),
        pltpu.SemaphoreType.DMA,
    ]
    if with_deg:
        outs.append(jax.ShapeDtypeStruct((NC, TOT, DEG_W), jnp.float32))
        scratch += [
            pltpu.VMEM((LANE, DEG_W), jnp.float32),
            pltpu.VMEM_SHARED((TOT, DEG_W), jnp.float32),
        ]

        def body(y, srcc, dstc, zer64, zer16, ones16, agg_out, deg_out,
                 src_v, dst_v, rows_v, acc, sem, ones_v, accd):
            _seg_body(y, srcc, dstc, zer64, agg_out, src_v, dst_v, rows_v,
                      acc, sem, deg=(zer16, ones16, deg_out, ones_v, accd))
    else:
        def body(y, srcc, dstc, zer64, agg_out,
                 src_v, dst_v, rows_v, acc, sem):
            _seg_body(y, srcc, dstc, zer64, agg_out, src_v, dst_v, rows_v,
                      acc, sem, deg=None)

    return pl.kernel(body, out_type=tuple(outs) if with_deg else outs[0],
                     mesh=mesh, scratch_types=scratch,
                     compiler_params=pltpu.CompilerParams(
                         use_tc_tiling_on_sc=False))


@functools.cache
def _get_seg_sum(with_deg):
    return _make_seg_sum(with_deg)


def _seg_sum_deg(y, srcc, dstc, zer64, zer16, ones16):
    return _get_seg_sum(True)(y, srcc, dstc, zer64, zer16, ones16)


def _seg_sum(y, srcc, dstc, zer64):
    return _get_seg_sum(False)(y, srcc, dstc, zer64)


def _mm_dual_body(x_ref, wl_ref, wp_ref, o_ref):
    w = jnp.where(pl.program_id(0) < NS, wl_ref[...], wp_ref[...])
    o_ref[...] = jnp.dot(x_ref[...], w, preferred_element_type=jnp.float32)


def _mm_dual(x, wl, wp):
    blk = TOT // 32
    return pl.pallas_call(
        _mm_dual_body,
        grid=(32,),
        in_specs=[
            pl.BlockSpec((blk, DIN), lambda i: (i, 0)),
            pl.BlockSpec((DIN, EMB), lambda i: (0, 0)),
            pl.BlockSpec((DIN, EMB), lambda i: (0, 0)),
        ],
        out_specs=pl.BlockSpec((blk, EMB), lambda i: (i, 0)),
        out_shape=jax.ShapeDtypeStruct((TOT, EMB), jnp.float32),
    )(x, wl, wp)


def _combine_body(parts_ref, deg_ref, bli_ref, bpi_ref, wl_ref, wp_ref,
                  blo_ref, bpo_ref, o_ref):
    sel = pl.program_id(0) < NS
    p = parts_ref[0] + parts_ref[1]
    deg = deg_ref[0, :, 0:1] + deg_ref[1, :, 0:1]
    b_in = jnp.where(sel, bli_ref[...], bpi_ref[...])
    w = jnp.where(sel, wl_ref[...], wp_ref[...])
    b_out = jnp.where(sel, blo_ref[...], bpo_ref[...])
    h = jnp.maximum(p / jnp.maximum(deg, 1.0) + b_in, 0.0)
    o_ref[...] = jnp.dot(h, w, preferred_element_type=jnp.float32) + b_out


def _combine(parts, degp, bl_in, bp_in, wl, wp, bl_out, bp_out):
    blk = TOT // 32
    b2 = lambda b: b.reshape(1, EMB)
    return pl.pallas_call(
        _combine_body,
        grid=(32,),
        in_specs=[
            pl.BlockSpec((NC, blk, EMB), lambda i: (0, i, 0)),
            pl.BlockSpec((NC, blk, DEG_W), lambda i: (0, i, 0)),
            pl.BlockSpec((1, EMB), lambda i: (0, 0)),
            pl.BlockSpec((1, EMB), lambda i: (0, 0)),
            pl.BlockSpec((EMB, EMB), lambda i: (0, 0)),
            pl.BlockSpec((EMB, EMB), lambda i: (0, 0)),
            pl.BlockSpec((1, EMB), lambda i: (0, 0)),
            pl.BlockSpec((1, EMB), lambda i: (0, 0)),
        ],
        out_specs=pl.BlockSpec((blk, EMB), lambda i: (i, 0)),
        out_shape=jax.ShapeDtypeStruct((TOT, EMB), jnp.float32),
    )(parts, degp, b2(bl_in), b2(bp_in), wl, wp, b2(bl_out), b2(bp_out))


def _scores_body(a_ref, b_ref, o_ref):
    s = lax.dot_general(a_ref[...], b_ref[...], (((1,), (1,)), ((), ())),
                        preferred_element_type=jnp.float32)
    o_ref[...] = 1.0 / (1.0 + jnp.exp(-s * SCALE))


def _scores(ab):
    bm, bn = 400, 512
    return pl.pallas_call(
        _scores_body,
        grid=(N // bm, -(-N // bn)),
        in_specs=[
            pl.BlockSpec((bm, EMB), lambda i, j: (i, 0)),
            pl.BlockSpec((bn, EMB), lambda i, j: (j + REG // bn, 0)),
        ],
        out_specs=pl.BlockSpec((bm, bn), lambda i, j: (i, j)),
        out_shape=jax.ShapeDtypeStruct((N, N), jnp.float32),
    )(ab, ab)


def kernel(logic_x, phys_x, Wl1, bl1, Wl2, bl2, Wp1, bp1, Wp2, bp2,
           Was, bas, Wap, bap, logic_edge_index, phys_edge_index):
    f32 = jnp.float32
    pad_rows = jnp.zeros((REG - N, DIN), f32)
    cat_x = jnp.concatenate([logic_x, pad_rows, phys_x, pad_rows], axis=0)

    src = jnp.concatenate([
        logic_edge_index[0].astype(jnp.int32),
        phys_edge_index[0].astype(jnp.int32) + REG,
        jnp.zeros((E_PAD - E_TOT,), jnp.int32),
    ])
    dst = jnp.concatenate([
        logic_edge_index[1].astype(jnp.int32),
        phys_edge_index[1].astype(jnp.int32) + REG,
        jnp.full((E_PAD - E_TOT,), N, jnp.int32),
    ])
    srcc = src.reshape(TOT_CHUNKS, LANE)
    dstc = dst.reshape(TOT_CHUNKS, LANE)
    zer64 = jnp.zeros((ROWS_PER_TILE, EMB), f32)
    zer16 = jnp.zeros((ROWS_PER_TILE, DEG_W), f32)
    ones16 = jnp.ones((LANE, DEG_W), f32)
    zb = jnp.zeros((EMB,), f32)

    y1 = _mm_dual(cat_x, Wl1, Wp1)
    agg1, degp = _seg_sum_deg(y1, srcc, dstc, zer64, zer16, ones16)
    y2 = _combine(agg1, degp, bl1, bp1, Wl2, Wp2, zb, zb)
    agg2 = _seg_sum(y2, srcc, dstc, zer64)
    ab = _combine(agg2, degp, bl2, bp2, Was, Wap, bas, bap)
    return _scores(ab)


# speedup vs baseline: 3.7692x; 1.2128x over previous
import functools

import jax
import jax.numpy as jnp
from jax import lax
from jax.experimental import pallas as pl
from jax.experimental.pallas import tpu as pltpu
from jax.experimental.pallas import tpu_sc as plsc

N = 10000
DIN = 128
EMB = 64
REG = 10240
TOT = 2 * REG
NC, NS = 2, 16
ROWS_PER_TILE = TOT // NS
LANE = 128
E_TOT = 320000 + 160000
CHUNKS_PER_TILE = 120
QC = CHUNKS_PER_TILE // 4
TOT_CHUNKS = NC * NS * CHUNKS_PER_TILE
E_PAD = TOT_CHUNKS * LANE
DEG_W = 16
SCALE = EMB ** (-0.5)


def _seg_body(y, idx2, zer64, agg_out, idx_v, rows0, rows1, acc, sem0, sem1,
              deg=None):
    cid = lax.axis_index("c")
    sid = lax.axis_index("s")
    w = cid * NS + sid
    c2 = 2 * CHUNKS_PER_TILE
    row0 = sid * ROWS_PER_TILE
    pltpu.sync_copy(zer64, acc.at[pl.ds(row0, ROWS_PER_TILE)])
    if deg is not None:
        zer16, ones16, deg_out, ones_v, accd = deg
        pltpu.sync_copy(zer16, accd.at[pl.ds(row0, ROWS_PER_TILE)])
        pltpu.sync_copy(ones16, ones_v)
    plsc.subcore_barrier()

    def body2(jj, carry):
        j2 = jj * 4
        pltpu.async_copy(y.at[idx_v.at[j2 + 2]], rows1, sem1)
        pltpu.make_async_copy(y.at[idx_v.at[j2]], rows0, sem0).wait()
        pltpu.sync_copy(rows0, acc.at[idx_v.at[j2 + 1]], add=True)
        if deg is not None:
            pltpu.sync_copy(ones_v, accd.at[idx_v.at[j2 + 1]], add=True)

        @pl.when(jj * 2 + 2 < QC)
        def _():
            pltpu.async_copy(y.at[idx_v.at[j2 + 4]], rows0, sem0)

        pltpu.make_async_copy(y.at[idx_v.at[j2 + 2]], rows1, sem1).wait()
        pltpu.sync_copy(rows1, acc.at[idx_v.at[j2 + 3]], add=True)
        if deg is not None:
            pltpu.sync_copy(ones_v, accd.at[idx_v.at[j2 + 3]], add=True)
        return carry

    for h in range(CHUNKS_PER_TILE // QC):
        pltpu.sync_copy(idx2.at[pl.ds(w * c2 + h * 2 * QC, 2 * QC)], idx_v)
        pltpu.async_copy(y.at[idx_v.at[0]], rows0, sem0)
        lax.fori_loop(0, QC // 2, body2, 0)
    plsc.subcore_barrier()
    pltpu.sync_copy(acc.at[pl.ds(row0, ROWS_PER_TILE)],
                    agg_out.at[cid, pl.ds(row0, ROWS_PER_TILE)])
    if deg is not None:
        pltpu.sync_copy(accd.at[pl.ds(row0, ROWS_PER_TILE)],
                        deg_out.at[cid, pl.ds(row0, ROWS_PER_TILE)])


def _make_seg_sum(with_deg):
    mesh = plsc.VectorSubcoreMesh(core_axis_name="c", subcore_axis_name="s")
    outs = [jax.ShapeDtypeStruct((NC, TOT, EMB), jnp.float32)]
    scratch = [
        pltpu.VMEM((2 * QC, LANE), jnp.int32),
        pltpu.VMEM((LANE, EMB), jnp.float32),
        pltpu.VMEM((LANE, EMB), jnp.float32),
        pltpu.VMEM_SHARED((TOT, EMB), jnp.float32),
        pltpu.SemaphoreType.DMA,
        pltpu.SemaphoreType.DMA,
    ]
    if with_deg:
        outs.append(jax.ShapeDtypeStruct((NC, TOT, DEG_W), jnp.float32))
        scratch += [
            pltpu.VMEM((LANE, DEG_W), jnp.float32),
            pltpu.VMEM_SHARED((TOT, DEG_W), jnp.float32),
        ]

        def body(y, idx2, zer64, zer16, ones16, agg_out, deg_out,
                 idx_v, rows0, rows1, acc, sem0, sem1, ones_v, accd):
            _seg_body(y, idx2, zer64, agg_out, idx_v, rows0, rows1, acc,
                      sem0, sem1,
                      deg=(zer16, ones16, deg_out, ones_v, accd))
    else:
        def body(y, idx2, zer64, agg_out,
                 idx_v, rows0, rows1, acc, sem0, sem1):
            _seg_body(y, idx2, zer64, agg_out, idx_v, rows0, rows1, acc,
                      sem0, sem1, deg=None)

    return pl.kernel(body, out_type=tuple(outs) if with_deg else outs[0],
                     mesh=mesh, scratch_types=scratch,
                     compiler_params=pltpu.CompilerParams(
                         use_tc_tiling_on_sc=False))


@functools.cache
def _get_seg_sum(with_deg):
    return _make_seg_sum(with_deg)


def _seg_sum_deg(y, idx2, zer64, zer16, ones16):
    return _get_seg_sum(True)(y, idx2, zer64, zer16, ones16)


def _seg_sum(y, idx2, zer64):
    return _get_seg_sum(False)(y, idx2, zer64)


def _mm_dual_body(x_ref, wl_ref, wp_ref, o_ref):
    w = jnp.where(pl.program_id(0) < NS, wl_ref[...], wp_ref[...])
    o_ref[...] = jnp.dot(x_ref[...], w, preferred_element_type=jnp.float32)


def _mm_dual(x, wl, wp):
    blk = TOT // 32
    return pl.pallas_call(
        _mm_dual_body,
        grid=(32,),
        in_specs=[
            pl.BlockSpec((blk, DIN), lambda i: (i, 0)),
            pl.BlockSpec((DIN, EMB), lambda i: (0, 0)),
            pl.BlockSpec((DIN, EMB), lambda i: (0, 0)),
        ],
        out_specs=pl.BlockSpec((blk, EMB), lambda i: (i, 0)),
        out_shape=jax.ShapeDtypeStruct((TOT, EMB), jnp.float32),
    )(x, wl, wp)


def _combine_body(parts_ref, deg_ref, bli_ref, bpi_ref, wl_ref, wp_ref,
                  blo_ref, bpo_ref, o_ref):
    sel = pl.program_id(0) < NS
    p = parts_ref[0] + parts_ref[1]
    deg = deg_ref[0, :, 0:1] + deg_ref[1, :, 0:1]
    b_in = jnp.where(sel, bli_ref[...], bpi_ref[...])
    w = jnp.where(sel, wl_ref[...], wp_ref[...])
    b_out = jnp.where(sel, blo_ref[...], bpo_ref[...])
    h = jnp.maximum(p / jnp.maximum(deg, 1.0) + b_in, 0.0)
    o_ref[...] = jnp.dot(h, w, preferred_element_type=jnp.float32) + b_out


def _combine(parts, degp, bl_in, bp_in, wl, wp, bl_out, bp_out):
    blk = TOT // 32
    b2 = lambda b: b.reshape(1, EMB)
    return pl.pallas_call(
        _combine_body,
        grid=(32,),
        in_specs=[
            pl.BlockSpec((NC, blk, EMB), lambda i: (0, i, 0)),
            pl.BlockSpec((NC, blk, DEG_W), lambda i: (0, i, 0)),
            pl.BlockSpec((1, EMB), lambda i: (0, 0)),
            pl.BlockSpec((1, EMB), lambda i: (0, 0)),
            pl.BlockSpec((EMB, EMB), lambda i: (0, 0)),
            pl.BlockSpec((EMB, EMB), lambda i: (0, 0)),
            pl.BlockSpec((1, EMB), lambda i: (0, 0)),
            pl.BlockSpec((1, EMB), lambda i: (0, 0)),
        ],
        out_specs=pl.BlockSpec((blk, EMB), lambda i: (i, 0)),
        out_shape=jax.ShapeDtypeStruct((TOT, EMB), jnp.float32),
    )(parts, degp, b2(bl_in), b2(bp_in), wl, wp, b2(bl_out), b2(bp_out))


def _scores_body(a_ref, b_ref, o_ref):
    s = lax.dot_general(a_ref[...], b_ref[...], (((1,), (1,)), ((), ())),
                        preferred_element_type=jnp.float32)
    o_ref[...] = 1.0 / (1.0 + jnp.exp(-s * SCALE))


def _scores(ab):
    bm, bn = 1000, 512
    return pl.pallas_call(
        _scores_body,
        grid=(N // bm, -(-N // bn)),
        in_specs=[
            pl.BlockSpec((bm, EMB), lambda i, j: (i, 0)),
            pl.BlockSpec((bn, EMB), lambda i, j: (j + REG // bn, 0)),
        ],
        out_specs=pl.BlockSpec((bm, bn), lambda i, j: (i, j)),
        out_shape=jax.ShapeDtypeStruct((N, N), jnp.float32),
    )(ab, ab)


def kernel(logic_x, phys_x, Wl1, bl1, Wl2, bl2, Wp1, bp1, Wp2, bp2,
           Was, bas, Wap, bap, logic_edge_index, phys_edge_index):
    f32 = jnp.float32
    pad_rows = jnp.zeros((REG - N, DIN), f32)
    cat_x = jnp.concatenate([logic_x, pad_rows, phys_x, pad_rows], axis=0)

    src = jnp.concatenate([
        logic_edge_index[0].astype(jnp.int32),
        phys_edge_index[0].astype(jnp.int32) + REG,
        jnp.zeros((E_PAD - E_TOT,), jnp.int32),
    ])
    dst = jnp.concatenate([
        logic_edge_index[1].astype(jnp.int32),
        phys_edge_index[1].astype(jnp.int32) + REG,
        N + (jnp.arange(E_PAD - E_TOT, dtype=jnp.int32) % (REG - N)),
    ])
    nw = NC * NS
    idx2 = jnp.stack([src.reshape(nw, CHUNKS_PER_TILE, LANE),
                      dst.reshape(nw, CHUNKS_PER_TILE, LANE)],
                     axis=2).reshape(nw * CHUNKS_PER_TILE * 2, LANE)
    zer64 = jnp.zeros((ROWS_PER_TILE, EMB), f32)
    zer16 = jnp.zeros((ROWS_PER_TILE, DEG_W), f32)
    ones16 = jnp.ones((LANE, DEG_W), f32)
    zb = jnp.zeros((EMB,), f32)

    y1 = _mm_dual(cat_x, Wl1, Wp1)
    agg1, degp = _seg_sum_deg(y1, idx2, zer64, zer16, ones16)
    y2 = _combine(agg1, degp, bl1, bp1, Wl2, Wp2, zb, zb)
    agg2 = _seg_sum(y2, idx2, zer64)
    ab = _combine(agg2, degp, bl2, bp2, Was, Wap, bas, bap)
    return _scores(ab)
